# baseline (speedup 1.0000x reference)
import functools

import jax
import jax.numpy as jnp
from jax import lax
from jax.experimental import pallas as pl
from jax.experimental.pallas import tpu as pltpu
from jax.experimental.pallas import tpu_sc as plsc

NUM_BIOMES = 11
EMBED_DIM = 64
BATCH = 16384

_info = plsc.get_sparse_core_info()
_NC = _info.num_cores
_NS = _info.num_subcores
_NW = _NC * _NS
_BPW = BATCH // _NW

_mesh = plsc.VectorSubcoreMesh(core_axis_name="c", subcore_axis_name="s")


@functools.partial(
    pl.kernel,
    mesh=_mesh,
    out_type=jax.ShapeDtypeStruct((BATCH, EMBED_DIM), jnp.float32),
    scratch_types=[
        pltpu.VMEM((_BPW,), jnp.int32),
        pltpu.VMEM((_BPW, EMBED_DIM), jnp.float32),
        pltpu.SemaphoreType.DMA,
    ],
    compiler_params=pltpu.CompilerParams(use_tc_tiling_on_sc=False),
)
def _emb_lookup(table_hbm, idx_hbm, out_hbm, idx_v, rows_v, sem):
    wid = lax.axis_index("s") * _NC + lax.axis_index("c")
    base = wid * _BPW
    pltpu.sync_copy(idx_hbm.at[pl.ds(base, _BPW)], idx_v)
    pltpu.async_copy(table_hbm.at[idx_v], rows_v, sem).wait()
    pltpu.sync_copy(rows_v, out_hbm.at[pl.ds(base, _BPW)])


def kernel(prompts, table):
    return _emb_lookup(table, prompts.astype(jnp.int32))


# speedup vs baseline: 1.2855x; 1.2855x over previous
import functools

import jax
import jax.numpy as jnp
from jax import lax
from jax.experimental import pallas as pl
from jax.experimental.pallas import tpu as pltpu
from jax.experimental.pallas import tpu_sc as plsc

NUM_BIOMES = 11
EMBED_DIM = 64
BATCH = 16384

_info = plsc.get_sparse_core_info()
_NC = _info.num_cores
_NS = _info.num_subcores
_NW = _NC * _NS
_BPW = BATCH // _NW
_L = 16

_mesh = plsc.VectorSubcoreMesh(core_axis_name="c", subcore_axis_name="s")


@functools.partial(
    pl.kernel,
    mesh=_mesh,
    out_type=jax.ShapeDtypeStruct((BATCH * EMBED_DIM,), jnp.float32),
    scratch_types=[
        pltpu.VMEM((NUM_BIOMES * EMBED_DIM,), jnp.float32),
        pltpu.VMEM((_BPW,), jnp.int32),
        pltpu.VMEM((_BPW * EMBED_DIM,), jnp.float32),
    ],
    compiler_params=pltpu.CompilerParams(
        use_tc_tiling_on_sc=False, needs_layout_passes=False
    ),
)
def _emb_lookup(table_hbm, idx_hbm, out_hbm, table_v, idx_v, rows_v):
    wid = lax.axis_index("s") * _NC + lax.axis_index("c")
    base = wid * _BPW
    pltpu.sync_copy(table_hbm, table_v)
    pltpu.sync_copy(idx_hbm.at[pl.ds(base, _BPW)], idx_v)

    lane = lax.iota(jnp.int32, _L)

    @pl.loop(0, _BPW // _L)
    def _group(g):
        pvec = idx_v[pl.ds(g * _L, _L)]
        src0 = pvec * EMBED_DIM
        dst0 = (lane + g * _L) * EMBED_DIM
        for j in range(EMBED_DIM):
            val = plsc.load_gather(table_v, [src0 + j])
            plsc.store_scatter(rows_v, [dst0 + j], val)

    pltpu.sync_copy(rows_v, out_hbm.at[pl.ds(base * EMBED_DIM, _BPW * EMBED_DIM)])


def kernel(prompts, table):
    flat = _emb_lookup(table.reshape(-1), prompts.astype(jnp.int32))
    return flat.reshape(BATCH, EMBED_DIM)


# speedup vs baseline: 1.6978x; 1.3207x over previous
import functools

import jax
import jax.numpy as jnp
from jax import lax
from jax.experimental import pallas as pl
from jax.experimental.pallas import tpu as pltpu
from jax.experimental.pallas import tpu_sc as plsc

NUM_BIOMES = 11
EMBED_DIM = 64
BATCH = 16384

_info = plsc.get_sparse_core_info()
_NC = _info.num_cores
_NS = _info.num_subcores
_NW = _NC * _NS
_BPW = BATCH // _NW
_L = 16
_GROUPS = _BPW // _L
_CHUNKG = 8
_NCHUNK = _GROUPS // _CHUNKG
_DEPTH = 8

_mesh = plsc.VectorSubcoreMesh(core_axis_name="c", subcore_axis_name="s")


@functools.partial(
    pl.kernel,
    mesh=_mesh,
    out_type=jax.ShapeDtypeStruct((BATCH * EMBED_DIM,), jnp.float32),
    scratch_types=[
        pltpu.VMEM((NUM_BIOMES * EMBED_DIM,), jnp.float32),
        pltpu.VMEM((_BPW,), jnp.int32),
        pltpu.VMEM((_BPW * EMBED_DIM,), jnp.float32),
        pltpu.SemaphoreType.DMA,
    ],
    compiler_params=pltpu.CompilerParams(
        use_tc_tiling_on_sc=False, needs_layout_passes=False
    ),
)
def _emb_lookup(table_hbm, idx_hbm, out_hbm, table_v, idx_v, rows_v, sem):
    wid = lax.axis_index("s") * _NC + lax.axis_index("c")
    base = wid * _BPW
    pltpu.sync_copy(table_hbm, table_v)
    pltpu.sync_copy(idx_hbm.at[pl.ds(base, _BPW)], idx_v)

    lane = lax.iota(jnp.int32, _L)
    chunk_elems = _CHUNKG * _L * EMBED_DIM
    copies = []
    for c in range(_NCHUNK):

        @plsc.parallel_loop(c * _CHUNKG, (c + 1) * _CHUNKG, unroll=2)
        def _group(g):
            pvec = idx_v[pl.ds(g * _L, _L)]
            src0 = pvec * EMBED_DIM
            dst0 = (lane + g * _L) * EMBED_DIM
            vals = {}
            for j in range(EMBED_DIM + _DEPTH):
                if j < EMBED_DIM:
                    vals[j] = plsc.load_gather(table_v, [src0 + j])
                if j >= _DEPTH:
                    plsc.store_scatter(rows_v, [dst0 + (j - _DEPTH)], vals.pop(j - _DEPTH))

        cp = pltpu.make_async_copy(
            rows_v.at[pl.ds(c * chunk_elems, chunk_elems)],
            out_hbm.at[pl.ds(base * EMBED_DIM + c * chunk_elems, chunk_elems)],
            sem,
        )
        cp.start()
        copies.append(cp)

    for cp in copies:
        cp.wait()


def kernel(prompts, table):
    flat = _emb_lookup(table.reshape(-1), prompts.astype(jnp.int32))
    return flat.reshape(BATCH, EMBED_DIM)


# speedup vs baseline: 2.1469x; 1.2645x over previous
import functools

import jax
import jax.numpy as jnp
from jax import lax
from jax.experimental import pallas as pl
from jax.experimental.pallas import tpu as pltpu
from jax.experimental.pallas import tpu_sc as plsc

NUM_BIOMES = 11
EMBED_DIM = 64
BATCH = 16384

_info = plsc.get_sparse_core_info()
_NC = _info.num_cores
_NS = _info.num_subcores
_NW = _NC * _NS
_BPW = BATCH // _NW
_L = 16
_CHUNKR = 128
_NCHUNK = _BPW // _CHUNKR

_mesh = plsc.VectorSubcoreMesh(core_axis_name="c", subcore_axis_name="s")


@functools.partial(
    pl.kernel,
    mesh=_mesh,
    out_type=jax.ShapeDtypeStruct((BATCH * EMBED_DIM,), jnp.float32),
    scratch_types=[
        pltpu.VMEM((NUM_BIOMES * EMBED_DIM,), jnp.float32),
        pltpu.VMEM((_BPW,), jnp.int32),
        pltpu.VMEM((_BPW * EMBED_DIM,), jnp.float32),
        pltpu.SemaphoreType.DMA,
    ],
    compiler_params=pltpu.CompilerParams(
        use_tc_tiling_on_sc=False, needs_layout_passes=False
    ),
)
def _emb_lookup(table_hbm, idx_hbm, out_hbm, table_v, idx_v, rows_v, sem):
    wid = lax.axis_index("s") * _NC + lax.axis_index("c")
    base = wid * _BPW
    pltpu.sync_copy(table_hbm, table_v)
    pltpu.sync_copy(idx_hbm.at[pl.ds(base, _BPW)], idx_v)

    chunk_elems = _CHUNKR * EMBED_DIM
    chunk_groups = _CHUNKR // _L
    copies = []
    for c in range(_NCHUNK):

        @plsc.parallel_loop(c * chunk_groups, (c + 1) * chunk_groups, unroll=1)
        def _group(g):
            pvec = idx_v[pl.ds(g * _L, _L)] * EMBED_DIM
            dst0 = g * _L * EMBED_DIM
            for r in range(_L):
                src = pvec[r]
                dst = dst0 + r * EMBED_DIM
                for k in range(EMBED_DIM // _L):
                    rows_v[pl.ds(dst + k * _L, _L)] = table_v[pl.ds(src + k * _L, _L)]

        cp = pltpu.make_async_copy(
            rows_v.at[pl.ds(c * chunk_elems, chunk_elems)],
            out_hbm.at[pl.ds(base * EMBED_DIM + c * chunk_elems, chunk_elems)],
            sem,
        )
        cp.start()
        copies.append(cp)

    for cp in copies:
        cp.wait()


def kernel(prompts, table):
    flat = _emb_lookup(table.reshape(-1), prompts.astype(jnp.int32))
    return flat.reshape(BATCH, EMBED_DIM)
